# Initial kernel scaffold; baseline (speedup 1.0000x reference)
import jax
import jax.numpy as jnp
from jax.experimental import pallas as pl


def kernel(pred, gt, freq_weights):
    raise NotImplementedError("write your pallas kernel here")


# speedup vs baseline: 11.4719x; 11.4719x over previous
import functools

import numpy as np
import ml_dtypes
import jax
import jax.numpy as jnp
from jax.experimental import pallas as pl
from jax.experimental.pallas import tpu as pltpu

_NUM_BINS = 16
_EPS = 1e-06


def _bf16_split_np(a):
    hi = a.astype(ml_dtypes.bfloat16)
    lo = (a - hi.astype(np.float32)).astype(ml_dtypes.bfloat16)
    return hi, lo


@functools.lru_cache(maxsize=None)
def _dft_mats(H, W, HU_P, VR_P):
    HU = H // 2 + 1
    VR = W // 2 + 1
    u = np.arange(HU)
    h = np.arange(H)
    ang1 = 2.0 * np.pi * ((np.outer(u, h) % H) / H)
    c1 = np.zeros((HU_P, H), np.float64)
    s1 = np.zeros((HU_P, H), np.float64)
    c1[:HU] = np.cos(ang1) / H
    s1[:HU] = np.sin(ang1) / H
    w = np.arange(W)
    v = np.arange(VR)
    ang2 = 2.0 * np.pi * ((np.outer(w, v) % W) / W)
    c2 = np.zeros((W, VR_P), np.float64)
    s2 = np.zeros((W, VR_P), np.float64)
    c2[:, :VR] = np.cos(ang2) / W
    s2[:, :VR] = np.sin(ang2) / W
    out = []
    for m in (c1, s1, c2, s2):
        out.extend(_bf16_split_np(m.astype(np.float32)))
    return tuple(out)


def _maxpool7(a, H, W):
    zr = jnp.zeros((3, W), a.dtype)
    m = a
    for d in (1, 2, 3):
        m = jnp.maximum(m, jnp.concatenate([zr[:d], a[:-d]], axis=0))
        m = jnp.maximum(m, jnp.concatenate([a[d:], zr[:d]], axis=0))
    zc = jnp.zeros((H, 3), a.dtype)
    r = m
    for d in (1, 2, 3):
        r = jnp.maximum(r, jnp.concatenate([zc[:, :d], m[:, :-d]], axis=1))
        r = jnp.maximum(r, jnp.concatenate([m[:, d:], zc[:, :d]], axis=1))
    return r


def _split(x):
    hi = x.astype(jnp.bfloat16)
    lo = (x - hi.astype(jnp.float32)).astype(jnp.bfloat16)
    return hi, lo


def _mm(ah, al, bh, bl):
    acc = jnp.dot(ah, bh, preferred_element_type=jnp.float32)
    if bl is not None:
        acc = acc + jnp.dot(ah, bl, preferred_element_type=jnp.float32)
    if al is not None:
        acc = acc + jnp.dot(al, bh, preferred_element_type=jnp.float32)
    return acc


def _make_k1(H, W, HU_P, VR_P):
    def k1(pred_ref, gt_ref, c1h, c1l, s1h, s1l, c2h, c2l, s2h, s2l,
           idx_ref, op_ref, og_ref):
        g = gt_ref[0]
        x = pred_ref[0]
        p = 1.0 / (1.0 + jnp.exp(-x))
        m1 = _maxpool7(g, H, W)
        m2 = _maxpool7(1.0 - g, H, W)
        bnd = jnp.clip(m1 + m2 - 1.0, 0.0, 1.0)
        bm_p = p * bnd
        bm_g = g * bnd

        ph, plo = _split(bm_p)
        gh = bm_g.astype(jnp.bfloat16)

        zr_p = _mm(c1h[...], c1l[...], ph, plo)
        zs_p = _mm(s1h[...], s1l[...], ph, plo)
        zr_g = _mm(c1h[...], c1l[...], gh, None)
        zs_g = _mm(s1h[...], s1l[...], gh, None)

        idx = idx_ref[...]
        rid = jax.lax.broadcasted_iota(jnp.int32, (HU_P, VR_P), 0)
        rmask = jnp.where((rid == 0) | (rid == H // 2), 0.0, 1.0)
        lane = jax.lax.broadcasted_iota(jnp.int32, (1, _NUM_BINS), 1)

        def profile_row(zr, zs):
            zrh, zrl = _split(zr)
            zsh, zsl = _split(zs)
            a = _mm(zrh, zrl, c2h[...], c2l[...])
            b = _mm(zsh, zsl, s2h[...], s2l[...])
            c = _mm(zrh, zrl, s2h[...], s2l[...])
            d = _mm(zsh, zsl, c2h[...], c2l[...])
            e = (a - b) ** 2 + (c + d) ** 2 + rmask * ((a + b) ** 2 + (d - c) ** 2)
            row = jnp.zeros((1, _NUM_BINS), jnp.float32)
            for bb in range(_NUM_BINS):
                s = jnp.sum(jnp.where(idx == bb, e, 0.0), keepdims=True)
                row = row + jnp.where(lane == bb, s, 0.0)
            return row

        op_ref[...] = profile_row(zr_p, zs_p).reshape(1, 1, _NUM_BINS)
        og_ref[...] = profile_row(zr_g, zs_g).reshape(1, 1, _NUM_BINS)

    return k1


def _k2(sp_ref, sg_ref, cnt_ref, fw_ref, out_ref):
    cnt = cnt_ref[...]
    pp = sp_ref[...] / cnt
    pp = pp / (jnp.sum(pp, axis=1, keepdims=True) + _EPS)
    pg = sg_ref[...] / cnt
    pg = pg / (jnp.sum(pg, axis=1, keepdims=True) + _EPS)
    diff = jnp.abs(pp - pg) * fw_ref[...]
    n = sp_ref.shape[0] * _NUM_BINS
    out_ref[...] = jnp.sum(diff, keepdims=True).reshape(1, 1) * (1.0 / n)


def kernel(pred, gt, freq_weights):
    B = pred.shape[0]
    H, W = pred.shape[2], pred.shape[3]
    HU = H // 2 + 1
    VR = W // 2 + 1
    HU_P = ((HU + 7) // 8) * 8
    VR_P = ((VR + 127) // 128) * 128

    p3 = pred.reshape(B, H, W)
    g3 = gt.reshape(B, H, W)

    fy = jnp.fft.fftfreq(H).astype(jnp.float32)[:, None]
    fx = jnp.fft.rfftfreq(W).astype(jnp.float32)[None, :]
    radius = jnp.sqrt(fy ** 2 + fx ** 2)
    radius = radius / jnp.maximum(radius.max(), _EPS)
    bin2d = jnp.minimum((radius * (_NUM_BINS - 1)).astype(jnp.int32),
                        _NUM_BINS - 1)
    counts = jnp.stack([jnp.sum(jnp.where(bin2d == b, 1.0, 0.0))
                        for b in range(_NUM_BINS)])
    counts = jnp.maximum(counts, 1.0).reshape(1, _NUM_BINS)
    idx_pad = jnp.pad(bin2d[:HU], ((0, HU_P - HU), (0, VR_P - VR)))

    mats = _dft_mats(H, W, HU_P, VR_P)

    def const(s):
        return pl.BlockSpec(s, lambda b: (0,) * len(s))

    mat_specs = [const((HU_P, H)), const((HU_P, H)), const((HU_P, H)),
                 const((HU_P, H)), const((W, VR_P)), const((W, VR_P)),
                 const((W, VR_P)), const((W, VR_P))]

    sums_p, sums_g = pl.pallas_call(
        _make_k1(H, W, HU_P, VR_P),
        grid=(B,),
        in_specs=[pl.BlockSpec((1, H, W), lambda b: (b, 0, 0)),
                  pl.BlockSpec((1, H, W), lambda b: (b, 0, 0)),
                  *mat_specs,
                  const((HU_P, VR_P))],
        out_specs=[pl.BlockSpec((1, 1, _NUM_BINS), lambda b: (b, 0, 0)),
                   pl.BlockSpec((1, 1, _NUM_BINS), lambda b: (b, 0, 0))],
        out_shape=[jax.ShapeDtypeStruct((B, 1, _NUM_BINS), jnp.float32),
                   jax.ShapeDtypeStruct((B, 1, _NUM_BINS), jnp.float32)],
        compiler_params=pltpu.CompilerParams(
            dimension_semantics=("parallel",)),
    )(p3, g3, *mats, idx_pad)

    loss = pl.pallas_call(
        _k2,
        out_shape=jax.ShapeDtypeStruct((1, 1), jnp.float32),
    )(sums_p.reshape(B, _NUM_BINS), sums_g.reshape(B, _NUM_BINS),
      counts, freq_weights.reshape(1, _NUM_BINS))
    return loss[0, 0]


# speedup vs baseline: 20.2533x; 1.7655x over previous
import functools

import numpy as np
import ml_dtypes
import jax
import jax.numpy as jnp
from jax.experimental import pallas as pl
from jax.experimental.pallas import tpu as pltpu

_NUM_BINS = 16
_EPS = 1e-06


def _bf16_split_np(a):
    hi = a.astype(ml_dtypes.bfloat16)
    lo = (a - hi.astype(np.float32)).astype(ml_dtypes.bfloat16)
    return hi, lo


@functools.lru_cache(maxsize=None)
def _dft_mats(H, W, HU_P, VR_P):
    HU = H // 2 + 1
    VR = W // 2 + 1
    u = np.arange(HU)
    h = np.arange(H)
    ang1 = 2.0 * np.pi * ((np.outer(u, h) % H) / H)
    c1 = np.zeros((HU_P, H), np.float64)
    s1 = np.zeros((HU_P, H), np.float64)
    c1[:HU] = np.cos(ang1) / H
    s1[:HU] = np.sin(ang1) / H
    w = np.arange(W)
    v = np.arange(VR)
    ang2 = 2.0 * np.pi * ((np.outer(w, v) % W) / W)
    c2 = np.zeros((W, VR_P), np.float64)
    s2 = np.zeros((W, VR_P), np.float64)
    c2[:, :VR] = np.cos(ang2) / W
    s2[:, :VR] = np.sin(ang2) / W
    out = []
    for m in (c1, s1, c2, s2):
        out.extend(_bf16_split_np(m.astype(np.float32)))
    return tuple(out)


@functools.lru_cache(maxsize=None)
def _pool_mats(H):
    i = np.arange(H)
    band = (np.abs(i[:, None] - i[None, :]) <= 3).astype(ml_dtypes.bfloat16)
    wr = np.minimum(i + 3, H - 1) - np.maximum(i - 3, 0) + 1
    w2 = np.asarray(np.outer(wr, wr), np.float32)
    return band, w2


def _mm(ah, al, bh, bl):
    acc = jnp.dot(ah, bh, preferred_element_type=jnp.float32)
    if bl is not None:
        acc = acc + jnp.dot(ah, bl, preferred_element_type=jnp.float32)
    if al is not None:
        acc = acc + jnp.dot(al, bh, preferred_element_type=jnp.float32)
    return acc


def _make_k1(H, W, HU_P, VR_P):
    def k1(pred_ref, gt_ref, b7_ref, w2_ref, c1h, c1l, s1h, s1l,
           c2h, c2l, s2h, s2l, idx_ref, op_ref, og_ref):
        g = gt_ref[0]
        x = pred_ref[0]
        p = 1.0 / (1.0 + jnp.exp(-x))
        b7 = b7_ref[...]
        rowc = jnp.dot(b7, g.astype(jnp.bfloat16),
                       preferred_element_type=jnp.float32)
        cnt = jnp.dot(rowc.astype(jnp.bfloat16), b7,
                      preferred_element_type=jnp.float32)
        bnd = jnp.where((cnt > 0.0) & (cnt < w2_ref[...]), 1.0, 0.0)
        bm_p = p * bnd
        bm_g = g * bnd

        ph = bm_p.astype(jnp.bfloat16)
        gh = bm_g.astype(jnp.bfloat16)

        zr_p = _mm(c1h[...], c1l[...], ph, None)
        zs_p = _mm(s1h[...], s1l[...], ph, None)
        zr_g = _mm(c1h[...], c1l[...], gh, None)
        zs_g = _mm(s1h[...], s1l[...], gh, None)

        idx = idx_ref[...]
        rid = jax.lax.broadcasted_iota(jnp.int32, (HU_P, VR_P), 0)
        rmask = jnp.where((rid == 0) | (rid == H // 2), 0.0, 1.0)
        lane = jax.lax.broadcasted_iota(jnp.int32, (1, _NUM_BINS), 1)

        def energies(zr, zs):
            zrh = zr.astype(jnp.bfloat16)
            zsh = zs.astype(jnp.bfloat16)
            a = _mm(zrh, None, c2h[...], c2l[...])
            b = _mm(zsh, None, s2h[...], s2l[...])
            c = _mm(zrh, None, s2h[...], s2l[...])
            d = _mm(zsh, None, c2h[...], c2l[...])
            return (a - b) ** 2 + (c + d) ** 2 + rmask * ((a + b) ** 2 + (d - c) ** 2)

        e_p = energies(zr_p, zs_p)
        e_g = energies(zr_g, zs_g)
        rowp = jnp.zeros((1, _NUM_BINS), jnp.float32)
        rowg = jnp.zeros((1, _NUM_BINS), jnp.float32)
        for bb in range(_NUM_BINS):
            m = idx == bb
            onehot = lane == bb
            sp = jnp.sum(jnp.where(m, e_p, 0.0), keepdims=True)
            sg = jnp.sum(jnp.where(m, e_g, 0.0), keepdims=True)
            rowp = rowp + jnp.where(onehot, sp, 0.0)
            rowg = rowg + jnp.where(onehot, sg, 0.0)

        op_ref[...] = rowp.reshape(1, 1, _NUM_BINS)
        og_ref[...] = rowg.reshape(1, 1, _NUM_BINS)

    return k1


def _k2(sp_ref, sg_ref, cnt_ref, fw_ref, out_ref):
    cnt = cnt_ref[...]
    pp = sp_ref[...] / cnt
    pp = pp / (jnp.sum(pp, axis=1, keepdims=True) + _EPS)
    pg = sg_ref[...] / cnt
    pg = pg / (jnp.sum(pg, axis=1, keepdims=True) + _EPS)
    diff = jnp.abs(pp - pg) * fw_ref[...]
    n = sp_ref.shape[0] * _NUM_BINS
    out_ref[...] = jnp.sum(diff, keepdims=True).reshape(1, 1) * (1.0 / n)


def kernel(pred, gt, freq_weights):
    B = pred.shape[0]
    H, W = pred.shape[2], pred.shape[3]
    HU = H // 2 + 1
    VR = W // 2 + 1
    HU_P = ((HU + 7) // 8) * 8
    VR_P = ((VR + 127) // 128) * 128

    p3 = pred.reshape(B, H, W)
    g3 = gt.reshape(B, H, W)

    fy = jnp.fft.fftfreq(H).astype(jnp.float32)[:, None]
    fx = jnp.fft.rfftfreq(W).astype(jnp.float32)[None, :]
    radius = jnp.sqrt(fy ** 2 + fx ** 2)
    radius = radius / jnp.maximum(radius.max(), _EPS)
    bin2d = jnp.minimum((radius * (_NUM_BINS - 1)).astype(jnp.int32),
                        _NUM_BINS - 1)
    counts = jnp.stack([jnp.sum(jnp.where(bin2d == b, 1.0, 0.0))
                        for b in range(_NUM_BINS)])
    counts = jnp.maximum(counts, 1.0).reshape(1, _NUM_BINS)
    idx_pad = jnp.pad(bin2d[:HU], ((0, HU_P - HU), (0, VR_P - VR)))

    mats = _dft_mats(H, W, HU_P, VR_P)
    band, w2 = _pool_mats(H)

    def const(s):
        return pl.BlockSpec(s, lambda b: (0,) * len(s))

    mat_specs = [const((HU_P, H)), const((HU_P, H)), const((HU_P, H)),
                 const((HU_P, H)), const((W, VR_P)), const((W, VR_P)),
                 const((W, VR_P)), const((W, VR_P))]

    sums_p, sums_g = pl.pallas_call(
        _make_k1(H, W, HU_P, VR_P),
        grid=(B,),
        in_specs=[pl.BlockSpec((1, H, W), lambda b: (b, 0, 0)),
                  pl.BlockSpec((1, H, W), lambda b: (b, 0, 0)),
                  const((H, H)), const((H, W)),
                  *mat_specs,
                  const((HU_P, VR_P))],
        out_specs=[pl.BlockSpec((1, 1, _NUM_BINS), lambda b: (b, 0, 0)),
                   pl.BlockSpec((1, 1, _NUM_BINS), lambda b: (b, 0, 0))],
        out_shape=[jax.ShapeDtypeStruct((B, 1, _NUM_BINS), jnp.float32),
                   jax.ShapeDtypeStruct((B, 1, _NUM_BINS), jnp.float32)],
        compiler_params=pltpu.CompilerParams(
            dimension_semantics=("parallel",)),
    )(p3, g3, band, w2, *mats, idx_pad)

    loss = pl.pallas_call(
        _k2,
        out_shape=jax.ShapeDtypeStruct((1, 1), jnp.float32),
    )(sums_p.reshape(B, _NUM_BINS), sums_g.reshape(B, _NUM_BINS),
      counts, freq_weights.reshape(1, _NUM_BINS))
    return loss[0, 0]
